# baseline (speedup 1.0000x reference)
import jax
import jax.numpy as jnp
from jax.experimental import pallas as pl
from jax.experimental.pallas import tpu as pltpu

N_TOK = 8192
D = 2048
BT = 512


def _router_tile(x_ref, w_hbm, x0_ref, x1_ref, out_ref, wv_ref, sem):
    @pl.when(pl.program_id(0) == 0)
    def _stage_w():
        cp = pltpu.make_async_copy(w_hbm, wv_ref, sem)
        cp.start()
        cp.wait()

    x = x_ref[...]
    w = wv_ref[...]
    s = jnp.dot(x, w, preferred_element_type=jnp.float32)
    d = s[:, 1:2] - s[:, 0:1]
    g1 = jax.nn.sigmoid(d)
    g0 = 1.0 - g1
    pick1 = d > 0.0
    w0 = jnp.where(pick1, 0.0, g0)
    w1 = jnp.where(pick1, g1, 0.0)
    x0_ref[...] = x * w0
    x1_ref[...] = x * w1
    out_ref[...] = x * (w0 + w1)


def kernel(x, W):
    grid = (N_TOK // BT,)
    shp = jax.ShapeDtypeStruct((N_TOK, D), x.dtype)
    x0, x1, out = pl.pallas_call(
        _router_tile,
        grid=grid,
        in_specs=[
            pl.BlockSpec((BT, D), lambda i: (i, 0)),
            pl.BlockSpec(memory_space=pltpu.HBM),
        ],
        out_specs=[
            pl.BlockSpec((BT, D), lambda i: (i, 0)),
            pl.BlockSpec((BT, D), lambda i: (i, 0)),
            pl.BlockSpec((BT, D), lambda i: (i, 0)),
        ],
        out_shape=[shp, shp, shp],
        scratch_shapes=[
            pltpu.VMEM((D, 2), jnp.float32),
            pltpu.SemaphoreType.DMA,
        ],
    )(x, W)
    return (x0, x1, out)


# speedup vs baseline: 1.0438x; 1.0438x over previous
import jax
import jax.numpy as jnp
from jax import lax
from jax.experimental import pallas as pl

N_TOK = 8192
D = 2048
BT = 512


def _router_tile(x_ref, wt_ref, x0_ref, x1_ref, out_ref):
    x = x_ref[...]
    wt = wt_ref[...]
    s = lax.dot_general(x, wt, (((1,), (1,)), ((), ())),
                        preferred_element_type=jnp.float32)
    d = s[:, 1:2] - s[:, 0:1]
    g1 = jax.nn.sigmoid(d)
    g0 = 1.0 - g1
    pick1 = d > 0.0
    w0 = jnp.where(pick1, 0.0, g0)
    w1 = jnp.where(pick1, g1, 0.0)
    x0_ref[...] = x * w0
    x1_ref[...] = x * w1
    out_ref[...] = x * (w0 + w1)


def kernel(x, W):
    wt = W.T
    grid = (N_TOK // BT,)
    shp = jax.ShapeDtypeStruct((N_TOK, D), x.dtype)
    x0, x1, out = pl.pallas_call(
        _router_tile,
        grid=grid,
        in_specs=[
            pl.BlockSpec((BT, D), lambda i: (i, 0)),
            pl.BlockSpec((2, D), lambda i: (0, 0)),
        ],
        out_specs=[
            pl.BlockSpec((BT, D), lambda i: (i, 0)),
            pl.BlockSpec((BT, D), lambda i: (i, 0)),
            pl.BlockSpec((BT, D), lambda i: (i, 0)),
        ],
        out_shape=[shp, shp, shp],
    )(x, wt)
    return (x0, x1, out)
